# Initial kernel scaffold; baseline (speedup 1.0000x reference)
import jax
import jax.numpy as jnp
from jax.experimental import pallas as pl


def kernel(x):
    raise NotImplementedError("write your pallas kernel here")


# speedup vs baseline: 79.1343x; 79.1343x over previous
import functools

import jax
import jax.numpy as jnp
from jax import lax
from jax.experimental import pallas as pl
from jax.experimental.pallas import tpu as pltpu
from jax.experimental.pallas import tpu_sc as plsc

_L = 16
_H = 224
_W = 224
_HW = _H * _W
_NIMG = 192
_NWORKERS = 32
_IMGS_PER_WORKER = _NIMG // _NWORKERS
_ROW_VECS = _W // 16


def _sc_hist_body(x_hbm, out_hbm, buf_a, buf_b, hist, sem_a, sem_b):
    wid = lax.axis_index("s") * 2 + lax.axis_index("c")
    img0 = wid * _IMGS_PER_WORKER

    bufs = (buf_a, buf_b)
    sems = (sem_a, sem_b)
    iota = lax.iota(jnp.int32, 16)
    ones = jnp.ones((16,), jnp.float32)
    zeros = jnp.zeros((16,), jnp.float32)
    v223 = jnp.full((16,), 223, jnp.int32)
    v0 = jnp.zeros((16,), jnp.int32)

    pending = pltpu.async_copy(x_hbm.at[img0], bufs[0], sems[0])

    for k in range(_IMGS_PER_WORKER):
        buf = bufs[k % 2]
        pending.wait()
        if k + 1 < _IMGS_PER_WORKER:
            pending = pltpu.async_copy(
                x_hbm.at[img0 + (k + 1)], bufs[(k + 1) % 2], sems[(k + 1) % 2]
            )

        def _zero(i, _):
            hist[pl.ds(i * 16, 16)] = zeros
            return _
        lax.fori_loop(0, 256 // 16, _zero, None)

        def _quant(i, _):
            v = buf[pl.ds(i * 16, 16)]
            q = (v * (_L - 1)).astype(jnp.int32).astype(jnp.float32)
            buf[pl.ds(i * 16, 16)] = q
            return _
        lax.fori_loop(0, _HW // 16, _quant, None)

        def _row(r, _):
            rowb = r * _W
            prevb = jnp.where(r == 0, (_H - 1) * _W, rowb - _W)

            def _blk(jb, _):
                cb = jb * 16
                c = cb + iota
                cm1 = jnp.where(c == 0, v223, c - 1)
                cp1 = jnp.where(c == v223, v0, c + 1)
                q = buf[pl.ds(rowb + cb, 16)]
                n90 = buf[pl.ds(prevb + cb, 16)]
                n0 = plsc.load_gather(buf, [rowb + cm1])
                n45 = plsc.load_gather(buf, [prevb + cp1])
                n135 = plsc.load_gather(buf, [prevb + cm1])
                q16 = q * float(_L)
                plsc.addupdate_scatter(hist, [(q16 + n0).astype(jnp.int32)], ones)
                plsc.addupdate_scatter(hist, [(q16 + n45).astype(jnp.int32)], ones)
                plsc.addupdate_scatter(hist, [(q16 + n90).astype(jnp.int32)], ones)
                plsc.addupdate_scatter(hist, [(q16 + n135).astype(jnp.int32)], ones)
                return _

            lax.fori_loop(0, _ROW_VECS, _blk, None)
            return _

        lax.fori_loop(0, _H, _row, None)

        pltpu.sync_copy(hist, out_hbm.at[img0 + k])


_sc_hist = functools.partial(
    pl.kernel,
    out_type=jax.ShapeDtypeStruct((_NIMG, _L * _L), jnp.float32),
    mesh=plsc.VectorSubcoreMesh(core_axis_name="c", subcore_axis_name="s"),
    scratch_types=[
        pltpu.VMEM((_HW,), jnp.float32),
        pltpu.VMEM((_HW,), jnp.float32),
        pltpu.VMEM((_L * _L,), jnp.float32),
        pltpu.SemaphoreType.DMA,
        pltpu.SemaphoreType.DMA,
    ],
    compiler_params=pltpu.CompilerParams(needs_layout_passes=False),
)(_sc_hist_body)


def _tc_entropy_body(hist_ref, out_ref):
    h = hist_ref[...]
    g = h / jnp.sum(h)
    e = -jnp.sum(g * jnp.log(g + 1e-10))
    out_ref[...] = jnp.full((1, _H, _W), e, jnp.float32)


def kernel(x):
    b, c, h, w = x.shape
    xr = x.reshape(_NIMG, _HW)
    hist = _sc_hist(xr).reshape(_NIMG, 1, _L * _L)
    ent = pl.pallas_call(
        _tc_entropy_body,
        grid=(_NIMG,),
        in_specs=[pl.BlockSpec((1, 1, _L * _L), lambda i: (i, 0, 0))],
        out_specs=pl.BlockSpec((1, _H, _W), lambda i: (i, 0, 0)),
        out_shape=jax.ShapeDtypeStruct((_NIMG, _H, _W), jnp.float32),
    )(hist)
    return ent.reshape(b, c, h, w)


# speedup vs baseline: 89.8694x; 1.1357x over previous
import functools

import jax
import jax.numpy as jnp
from jax import lax
from jax.experimental import pallas as pl
from jax.experimental.pallas import tpu as pltpu
from jax.experimental.pallas import tpu_sc as plsc

_L = 16
_H = 224
_W = 224
_HW = _H * _W
_NIMG = 192
_NWORKERS = 32
_IMGS_PER_WORKER = _NIMG // _NWORKERS
_ROW_VECS = _W // 16
_TC_BLK = 16


def _sc_hist_body(x_hbm, out_hbm, buf_a, buf_b, hist, sem_a, sem_b):
    wid = lax.axis_index("s") * 2 + lax.axis_index("c")
    img0 = wid * _IMGS_PER_WORKER

    bufs = (buf_a, buf_b)
    sems = (sem_a, sem_b)
    iota = lax.iota(jnp.int32, 16)
    ones = jnp.ones((16,), jnp.float32)
    zeros = jnp.zeros((16,), jnp.float32)
    cm1_first = jnp.where(iota == 0, jnp.full((16,), 223, jnp.int32), iota - 1)
    cp1_last = jnp.where(
        iota == 15, jnp.full((16,), -15, jnp.int32), iota + 1
    ) + (_W - 16)

    pending = pltpu.async_copy(x_hbm.at[img0], bufs[0], sems[0])

    for k in range(_IMGS_PER_WORKER):
        buf = bufs[k % 2]
        pending.wait()
        if k + 1 < _IMGS_PER_WORKER:
            pending = pltpu.async_copy(
                x_hbm.at[img0 + (k + 1)], bufs[(k + 1) % 2], sems[(k + 1) % 2]
            )

        def _zero(i, _):
            hist[pl.ds(i * 16, 16)] = zeros
            return _
        lax.fori_loop(0, 256 // 16, _zero, None)

        def _quant(v):
            return (v * (_L - 1)).astype(jnp.int32)

        def _scat(q16, n):
            plsc.addupdate_scatter(hist, [q16 + _quant(n)], ones)

        def _row(r, _):
            rowb = r * _W
            prevb = jnp.where(r == 0, (_H - 1) * _W, rowb - _W)
            for jb in range(_ROW_VECS):
                cb = jb * 16
                q16 = _quant(buf[pl.ds(rowb + cb, 16)]) * _L
                _scat(q16, buf[pl.ds(prevb + cb, 16)])
                if jb == 0:
                    n0 = plsc.load_gather(buf, [rowb + cm1_first])
                    n135 = plsc.load_gather(buf, [prevb + cm1_first])
                else:
                    n0 = buf[pl.ds(rowb + cb - 1, 16)]
                    n135 = buf[pl.ds(prevb + cb - 1, 16)]
                if jb == _ROW_VECS - 1:
                    n45 = plsc.load_gather(buf, [prevb + cp1_last])
                else:
                    n45 = buf[pl.ds(prevb + cb + 1, 16)]
                _scat(q16, n0)
                _scat(q16, n45)
                _scat(q16, n135)
            return _

        lax.fori_loop(0, _H, _row, None)

        pltpu.sync_copy(hist, out_hbm.at[img0 + k])


_sc_hist = functools.partial(
    pl.kernel,
    out_type=jax.ShapeDtypeStruct((_NIMG, _L * _L), jnp.float32),
    mesh=plsc.VectorSubcoreMesh(core_axis_name="c", subcore_axis_name="s"),
    scratch_types=[
        pltpu.VMEM((_HW,), jnp.float32),
        pltpu.VMEM((_HW,), jnp.float32),
        pltpu.VMEM((_L * _L,), jnp.float32),
        pltpu.SemaphoreType.DMA,
        pltpu.SemaphoreType.DMA,
    ],
    compiler_params=pltpu.CompilerParams(needs_layout_passes=False),
)(_sc_hist_body)


def _tc_entropy_body(hist_ref, out_ref):
    h = hist_ref[...]
    g = h / jnp.sum(h, axis=-1, keepdims=True)
    e = -jnp.sum(g * jnp.log(g + 1e-10), axis=-1)
    out_ref[...] = jnp.broadcast_to(e[:, :, None], (_TC_BLK, _H, _W))


def kernel(x):
    b, c, h, w = x.shape
    xr = x.reshape(_NIMG, _HW)
    hist = _sc_hist(xr).reshape(_NIMG, 1, _L * _L)
    ent = pl.pallas_call(
        _tc_entropy_body,
        grid=(_NIMG // _TC_BLK,),
        in_specs=[pl.BlockSpec((_TC_BLK, 1, _L * _L), lambda i: (i, 0, 0))],
        out_specs=pl.BlockSpec((_TC_BLK, _H, _W), lambda i: (i, 0, 0)),
        out_shape=jax.ShapeDtypeStruct((_NIMG, _H, _W), jnp.float32),
    )(hist)
    return ent.reshape(b, c, h, w)


# speedup vs baseline: 183.9275x; 2.0466x over previous
import functools

import jax
import jax.numpy as jnp
from jax import lax
from jax.experimental import pallas as pl
from jax.experimental.pallas import tpu as pltpu
from jax.experimental.pallas import tpu_sc as plsc

_L = 16
_H = 224
_W = 224
_HW = _H * _W
_NIMG = 192
_NWORKERS = 32
_IMGS_PER_WORKER = _NIMG // _NWORKERS
_ROW_VECS = _W // 16
_TC_BLK = 16


def _sc_hist_body(x_hbm, out_hbm, buf_a, buf_b, hist, sem_a, sem_b):
    wid = lax.axis_index("s") * 2 + lax.axis_index("c")
    img0 = wid * _IMGS_PER_WORKER

    bufs = (buf_a, buf_b)
    sems = (sem_a, sem_b)
    iota = lax.iota(jnp.int32, 16)
    ones = jnp.ones((16,), jnp.float32)
    zeros = jnp.zeros((16,), jnp.float32)
    cm1_first = jnp.where(iota == 0, jnp.full((16,), 223, jnp.int32), iota - 1)
    cp1_last = jnp.where(
        iota == 15, jnp.full((16,), -15, jnp.int32), iota + 1
    ) + (_W - 16)

    pending = pltpu.async_copy(x_hbm.at[img0], bufs[0], sems[0])

    for k in range(_IMGS_PER_WORKER):
        buf = bufs[k % 2]
        pending.wait()
        if k + 1 < _IMGS_PER_WORKER:
            pending = pltpu.async_copy(
                x_hbm.at[img0 + (k + 1)], bufs[(k + 1) % 2], sems[(k + 1) % 2]
            )

        def _zero(i, _):
            hist[pl.ds(i * 16, 16)] = zeros
            return _
        lax.fori_loop(0, 256 // 16, _zero, None)

        def _quant(v):
            return (v * (_L - 1)).astype(jnp.int32)

        def _row(r, _):
            rowb = r * _W
            prevb = jnp.where(r == 0, (_H - 1) * _W, rowb - _W)

            def _load(jb):
                cb = jb * 16
                q_raw = buf[pl.ds(rowb + cb, 16)]
                n90 = buf[pl.ds(prevb + cb, 16)]
                if jb == 0:
                    n0 = plsc.load_gather(buf, [rowb + cm1_first])
                    n135 = plsc.load_gather(buf, [prevb + cm1_first])
                else:
                    n0 = buf[pl.ds(rowb + cb - 1, 16)]
                    n135 = buf[pl.ds(prevb + cb - 1, 16)]
                if jb == _ROW_VECS - 1:
                    n45 = plsc.load_gather(buf, [prevb + cp1_last])
                else:
                    n45 = buf[pl.ds(prevb + cb + 1, 16)]
                return q_raw, n0, n45, n90, n135

            def _compute(vals):
                q_raw, n0, n45, n90, n135 = vals
                q16 = _quant(q_raw) * _L
                return tuple(q16 + _quant(n) for n in (n0, n45, n90, n135))

            def _scatter(idxs):
                for idx in idxs:
                    plsc.addupdate_scatter(hist, [idx], ones)

            vals = _load(0)
            idxs = _compute(vals)
            vals = _load(1)
            for jb in range(2, _ROW_VECS):
                _scatter(idxs)
                idxs = _compute(vals)
                vals = _load(jb)
            _scatter(idxs)
            idxs = _compute(vals)
            _scatter(idxs)
            return _

        lax.fori_loop(0, _H, _row, None)

        pltpu.sync_copy(hist, out_hbm.at[img0 + k])


_sc_hist = functools.partial(
    pl.kernel,
    out_type=jax.ShapeDtypeStruct((_NIMG, _L * _L), jnp.float32),
    mesh=plsc.VectorSubcoreMesh(core_axis_name="c", subcore_axis_name="s"),
    scratch_types=[
        pltpu.VMEM((_HW,), jnp.float32),
        pltpu.VMEM((_HW,), jnp.float32),
        pltpu.VMEM((_L * _L,), jnp.float32),
        pltpu.SemaphoreType.DMA,
        pltpu.SemaphoreType.DMA,
    ],
    compiler_params=pltpu.CompilerParams(needs_layout_passes=False),
)(_sc_hist_body)


def _tc_entropy_body(hist_ref, out_ref):
    h = hist_ref[...]
    g = h / jnp.sum(h, axis=-1, keepdims=True)
    e = -jnp.sum(g * jnp.log(g + 1e-10), axis=-1)
    out_ref[...] = jnp.broadcast_to(e[:, :, None], (_TC_BLK, _H, _W))


def kernel(x):
    b, c, h, w = x.shape
    xr = x.reshape(_NIMG, _HW)
    hist = _sc_hist(xr).reshape(_NIMG, 1, _L * _L)
    ent = pl.pallas_call(
        _tc_entropy_body,
        grid=(_NIMG // _TC_BLK,),
        in_specs=[pl.BlockSpec((_TC_BLK, 1, _L * _L), lambda i: (i, 0, 0))],
        out_specs=pl.BlockSpec((_TC_BLK, _H, _W), lambda i: (i, 0, 0)),
        out_shape=jax.ShapeDtypeStruct((_NIMG, _H, _W), jnp.float32),
    )(hist)
    return ent.reshape(b, c, h, w)


# speedup vs baseline: 211.3549x; 1.1491x over previous
import functools

import jax
import jax.numpy as jnp
from jax import lax
from jax.experimental import pallas as pl
from jax.experimental.pallas import tpu as pltpu
from jax.experimental.pallas import tpu_sc as plsc

_L = 16
_H = 224
_W = 224
_HW = _H * _W
_NIMG = 192
_NWORKERS = 32
_IMGS_PER_WORKER = _NIMG // _NWORKERS
_ROW_VECS = _W // 16
_QUNROLL = 8
_TC_BLK = 32


def _sc_hist_body(x_hbm, out_hbm, buf, qbuf, hist, sem):
    wid = lax.axis_index("s") * 2 + lax.axis_index("c")
    img0 = wid * _IMGS_PER_WORKER

    iota = lax.iota(jnp.int32, 16)
    ones = jnp.ones((16,), jnp.float32)
    zeros = jnp.zeros((16,), jnp.float32)
    cm1_first = jnp.where(iota == 0, jnp.full((16,), 223, jnp.int32), iota - 1)
    cp1_last = jnp.where(
        iota == 15, jnp.full((16,), -15, jnp.int32), iota + 1
    ) + (_W - 16)

    pending = pltpu.async_copy(x_hbm.at[img0], buf, sem)

    for k in range(_IMGS_PER_WORKER):
        pending.wait()

        def _quant_blk(i, _):
            base = i * (16 * _QUNROLL)
            vals = [buf[pl.ds(base + u * 16, 16)] for u in range(_QUNROLL)]
            qs = [(v * (_L - 1)).astype(jnp.int32) for v in vals]
            for u in range(_QUNROLL):
                qbuf[pl.ds(base + u * 16, 16)] = qs[u]
            return _
        lax.fori_loop(0, _HW // (16 * _QUNROLL), _quant_blk, None)

        if k + 1 < _IMGS_PER_WORKER:
            pending = pltpu.async_copy(x_hbm.at[img0 + (k + 1)], buf, sem)

        def _zero(i, _):
            hist[pl.ds(i * 16, 16)] = zeros
            return _
        lax.fori_loop(0, 256 // 16, _zero, None)

        def _rowpair(i, _):
            rowa = (2 * i) * _W
            rowbb = rowa + _W
            preva = jnp.where(i == 0, (_H - 1) * _W, rowa - _W)

            def _load(jb):
                cb = jb * 16
                qa = qbuf[pl.ds(rowa + cb, 16)]
                qb = qbuf[pl.ds(rowbb + cb, 16)]
                n90a = qbuf[pl.ds(preva + cb, 16)]
                if jb == 0:
                    n0a = plsc.load_gather(qbuf, [rowa + cm1_first])
                    n0b = plsc.load_gather(qbuf, [rowbb + cm1_first])
                    n135a = plsc.load_gather(qbuf, [preva + cm1_first])
                    n135b = plsc.load_gather(qbuf, [rowa + cm1_first])
                else:
                    n0a = qbuf[pl.ds(rowa + cb - 1, 16)]
                    n0b = qbuf[pl.ds(rowbb + cb - 1, 16)]
                    n135a = qbuf[pl.ds(preva + cb - 1, 16)]
                    n135b = qbuf[pl.ds(rowa + cb - 1, 16)]
                if jb == _ROW_VECS - 1:
                    n45a = plsc.load_gather(qbuf, [preva + cp1_last])
                    n45b = plsc.load_gather(qbuf, [rowa + cp1_last])
                else:
                    n45a = qbuf[pl.ds(preva + cb + 1, 16)]
                    n45b = qbuf[pl.ds(rowa + cb + 1, 16)]
                return qa, qb, n90a, n0a, n0b, n45a, n45b, n135a, n135b

            def _compute(vals):
                qa, qb, n90a, n0a, n0b, n45a, n45b, n135a, n135b = vals
                q16a = qa * _L
                q16b = qb * _L
                return (
                    q16a + n0a, q16a + n45a, q16a + n90a, q16a + n135a,
                    q16b + n0b, q16b + n45b, q16b + qa, q16b + n135b,
                )

            def _scatter(idxs):
                for idx in idxs:
                    plsc.addupdate_scatter(hist, [idx], ones)

            vals = _load(0)
            idxs = _compute(vals)
            vals = _load(1)
            for jb in range(2, _ROW_VECS):
                _scatter(idxs)
                idxs = _compute(vals)
                vals = _load(jb)
            _scatter(idxs)
            idxs = _compute(vals)
            _scatter(idxs)
            return _

        lax.fori_loop(0, _H // 2, _rowpair, None)

        pltpu.sync_copy(hist, out_hbm.at[img0 + k])


_sc_hist = functools.partial(
    pl.kernel,
    out_type=jax.ShapeDtypeStruct((_NIMG, _L * _L), jnp.float32),
    mesh=plsc.VectorSubcoreMesh(core_axis_name="c", subcore_axis_name="s"),
    scratch_types=[
        pltpu.VMEM((_HW,), jnp.float32),
        pltpu.VMEM((_HW,), jnp.int32),
        pltpu.VMEM((_L * _L,), jnp.float32),
        pltpu.SemaphoreType.DMA,
    ],
    compiler_params=pltpu.CompilerParams(needs_layout_passes=False),
)(_sc_hist_body)


def _tc_entropy_body(hist_ref, out_ref):
    h = hist_ref[...]
    g = h / jnp.sum(h, axis=-1, keepdims=True)
    e = -jnp.sum(g * jnp.log(g + 1e-10), axis=-1)
    out_ref[...] = jnp.broadcast_to(e[:, :, None], (_TC_BLK, _H, _W))


def kernel(x):
    b, c, h, w = x.shape
    xr = x.reshape(_NIMG, _HW)
    hist = _sc_hist(xr).reshape(_NIMG, 1, _L * _L)
    ent = pl.pallas_call(
        _tc_entropy_body,
        grid=(_NIMG // _TC_BLK,),
        in_specs=[pl.BlockSpec((_TC_BLK, 1, _L * _L), lambda i: (i, 0, 0))],
        out_specs=pl.BlockSpec((_TC_BLK, _H, _W), lambda i: (i, 0, 0)),
        out_shape=jax.ShapeDtypeStruct((_NIMG, _H, _W), jnp.float32),
    )(hist)
    return ent.reshape(b, c, h, w)
